# baseline (speedup 1.0000x reference)
import jax
import jax.numpy as jnp
from jax.experimental import pallas as pl
from jax.experimental.pallas import tpu as pltpu

_B = 1024
_L = 64
_BB = 256
_NBLK = _B // _BB
_CHUNKS = ((0, 2560), (2560, 2560), (5120, 2560), (7680, 2320))


def _body(dec_hbm, mov_hbm, cat_hbm, per_hbm, com_hbm,
          wd_ref, wm0, wm1, wm2, wm3, wc_ref, wp0, wp1, wp2, wp3, wco_ref,
          out_ref,
          bd_ref, bm0, bm1, bm2, bm3, bc_ref, bp0, bp1, bp2, bp3, bco_ref,
          sems):
    mov_bufs = (bm0, bm1, bm2, bm3)
    per_bufs = (bp0, bp1, bp2, bp3)
    wms = (wm0, wm1, wm2, wm3)
    wps = (wp0, wp1, wp2, wp3)

    chunks = (
        [(mov_bufs[j], mov_hbm, _CHUNKS[j][0], _CHUNKS[j][1], j)
         for j in range(4)]
        + [(per_bufs[j], per_hbm, _CHUNKS[j][0], _CHUNKS[j][1], 4 + j)
           for j in range(4)]
        + [(bc_ref, cat_hbm, 0, 1000, 8), (bco_ref, com_hbm, 0, 1000, 9)]
    )

    def start_block(b):
        slot = b % 2
        for buf, src, off, width, ci in chunks:
            pltpu.make_async_copy(
                src.at[pl.ds(b * _BB, _BB), pl.ds(off, width)],
                buf.at[slot],
                sems.at[slot, ci],
            ).start()

    pltpu.make_async_copy(dec_hbm, bd_ref, sems.at[0, 10]).start()
    start_block(0)

    dn = (((1,), (0,)), ((), ()))

    def mean_div(y, s):
        nz = s != 0.0
        return jnp.where(nz[:, None], y / jnp.where(nz, s, 1.0)[:, None], y)

    for b in range(_NBLK):
        if b + 1 < _NBLK:
            start_block(b + 1)
        slot = b % 2

        def chunk_dot(buf, src, off, width, ci, w_ref):
            pltpu.make_async_copy(
                src.at[pl.ds(b * _BB, _BB), pl.ds(off, width)],
                buf.at[slot],
                sems.at[slot, ci],
            ).wait()
            xb = buf[slot].astype(jnp.bfloat16)
            return jax.lax.dot_general(xb, w_ref[...], dn,
                                       preferred_element_type=jnp.float32)

        ym = None
        for (buf, src, off, width, ci), w_ref in zip(chunks[0:4], wms):
            p = chunk_dot(buf, src, off, width, ci, w_ref)
            ym = p if ym is None else ym + p
        yp = None
        for (buf, src, off, width, ci), w_ref in zip(chunks[4:8], wps):
            p = chunk_dot(buf, src, off, width, ci, w_ref)
            yp = p if yp is None else yp + p
        yc = chunk_dot(*chunks[8], wc_ref)
        yco = chunk_dot(*chunks[9], wco_ref)

        if b == 0:
            pltpu.make_async_copy(dec_hbm, bd_ref, sems.at[0, 10]).wait()
        xd = bd_ref[pl.ds(b * _BB, _BB), :].astype(jnp.bfloat16)
        yd = jax.lax.dot_general(xd, wd_ref[...], dn,
                                 preferred_element_type=jnp.float32)

        sd, sm, sc, sp, sco = (y[:, _L] for y in (yd, ym, yc, yp, yco))
        yd, ym, yc, yp, yco = (y[:, :_L] for y in (yd, ym, yc, yp, yco))

        yd = mean_div(yd, sd)
        yd = mean_div(yd, sm)
        yc = mean_div(yc, sc)
        yp = mean_div(yp, sp)
        yco = mean_div(yco, sco)

        out_ref[pl.ds(b * _BB, _BB), :] = jnp.concatenate(
            (yd, ym, yc, yp, yco), axis=1)


def _aug_t(W):
    wt = jnp.concatenate([W.T, jnp.ones((W.shape[1], 1), jnp.float32)],
                         axis=1)
    return wt.astype(jnp.bfloat16)


def kernel(decade_idxs, movie_idxs, category_idxs, person_idxs, company_idxs,
           W_decade, W_movie, W_category, W_person, W_company):
    wd = _aug_t(W_decade)
    wc = _aug_t(W_category)
    wco = _aug_t(W_company)
    wmt = _aug_t(W_movie)
    wpt = _aug_t(W_person)
    wms = [wmt[o:o + w] for o, w in _CHUNKS]
    wps = [wpt[o:o + w] for o, w in _CHUNKS]

    any_spec = pl.BlockSpec(memory_space=pl.ANY)

    def w_spec(k):
        return pl.BlockSpec((k, _L + 1), lambda: (0, 0))

    in_specs = (
        [any_spec] * 5
        + [w_spec(16)] + [w_spec(w) for _, w in _CHUNKS] + [w_spec(1000)]
        + [w_spec(w) for _, w in _CHUNKS] + [w_spec(1000)]
    )
    scratch_shapes = (
        [pltpu.VMEM((_B, 16), jnp.int32)]
        + [pltpu.VMEM((2, _BB, w), jnp.int32) for _, w in _CHUNKS]
        + [pltpu.VMEM((2, _BB, 1000), jnp.int32)]
        + [pltpu.VMEM((2, _BB, w), jnp.int32) for _, w in _CHUNKS]
        + [pltpu.VMEM((2, _BB, 1000), jnp.int32)]
        + [pltpu.SemaphoreType.DMA((2, 11))]
    )
    out = pl.pallas_call(
        _body,
        in_specs=in_specs,
        out_specs=pl.BlockSpec((_B, 5 * _L), lambda: (0, 0)),
        out_shape=jax.ShapeDtypeStruct((_B, 5 * _L), jnp.float32),
        scratch_shapes=scratch_shapes,
    )(decade_idxs, movie_idxs, category_idxs, person_idxs, company_idxs,
      wd, *wms, wc, *wps, wco)
    return out


# speedup vs baseline: 2.8862x; 2.8862x over previous
import jax
import jax.numpy as jnp
from jax.experimental import pallas as pl
from jax.experimental.pallas import tpu as pltpu

_B = 1024
_L = 64
_KBIG = 10000
_KC = 1000
_NSTEP = _KBIG // _KC


def _body(xd_ref, xm_ref, xc_ref, xp_ref, xco_ref,
          wd_ref, wm_ref, wc_ref, wp_ref, wco_ref,
          out_ref, ym_acc, yp_acc):
    step = pl.program_id(0)
    dn = (((1,), (0,)), ((), ()))

    def part(w_ref, x_ref):
        xb = x_ref[...].astype(jnp.bfloat16)
        return jax.lax.dot_general(w_ref[...], xb, dn,
                                   preferred_element_type=jnp.float32)

    pm = part(wm_ref.at[0], xm_ref)
    pp = part(wp_ref.at[0], xp_ref)

    @pl.when(step == 0)
    def _():
        ym_acc[...] = pm
        yp_acc[...] = pp

    @pl.when(step != 0)
    def _():
        ym_acc[...] += pm
        yp_acc[...] += pp

    @pl.when(step == _NSTEP - 1)
    def _():
        yd = part(wd_ref, xd_ref)
        yc = part(wc_ref, xc_ref)
        yco = part(wco_ref, xco_ref)
        ym = ym_acc[...]
        yp = yp_acc[...]

        def mean_div(y, s):
            nz = s != 0.0
            return jnp.where(nz, y / jnp.where(nz, s, 1.0), y)

        sd, sm, sc, sp, sco = (y[_L:_L + 1, :]
                               for y in (yd, ym, yc, yp, yco))
        yd, ym, yc, yp, yco = (y[:_L, :] for y in (yd, ym, yc, yp, yco))

        yd = mean_div(yd, sd)
        yd = mean_div(yd, sm)
        yc = mean_div(yc, sc)
        yp = mean_div(yp, sp)
        yco = mean_div(yco, sco)

        out_ref[...] = jnp.concatenate((yd, ym, yc, yp, yco), axis=0)


def _aug(W):
    wa = jnp.concatenate([W, jnp.ones((1, W.shape[1]), jnp.float32)], axis=0)
    return wa.astype(jnp.bfloat16)


def kernel(decade_idxs, movie_idxs, category_idxs, person_idxs, company_idxs,
           W_decade, W_movie, W_category, W_person, W_company):
    xd, xm, xc, xp, xco = (x.T for x in (
        decade_idxs, movie_idxs, category_idxs, person_idxs, company_idxs))
    wd, wm, wc, wp, wco = (_aug(W) for W in (
        W_decade, W_movie, W_category, W_person, W_company))
    la = _L + 1
    wm = wm.reshape(la, _NSTEP, _KC).transpose(1, 0, 2)
    wp = wp.reshape(la, _NSTEP, _KC).transpose(1, 0, 2)

    grid = (_NSTEP,)

    in_specs = [
        pl.BlockSpec((16, _B), lambda i: (0, 0)),
        pl.BlockSpec((_KC, _B), lambda i: (i, 0)),
        pl.BlockSpec((_KC, _B), lambda i: (0, 0)),
        pl.BlockSpec((_KC, _B), lambda i: (i, 0)),
        pl.BlockSpec((_KC, _B), lambda i: (0, 0)),
        pl.BlockSpec((la, 16), lambda i: (0, 0)),
        pl.BlockSpec((1, la, _KC), lambda i: (i, 0, 0)),
        pl.BlockSpec((la, _KC), lambda i: (0, 0)),
        pl.BlockSpec((1, la, _KC), lambda i: (i, 0, 0)),
        pl.BlockSpec((la, _KC), lambda i: (0, 0)),
    ]
    out_t = pl.pallas_call(
        _body,
        grid=grid,
        in_specs=in_specs,
        out_specs=pl.BlockSpec((5 * _L, _B), lambda i: (0, 0)),
        out_shape=jax.ShapeDtypeStruct((5 * _L, _B), jnp.float32),
        scratch_shapes=[pltpu.VMEM((la, _B), jnp.float32)] * 2,
    )(xd, xm, xc, xp, xco, wd, wm, wc, wp, wco)
    return out_t.T
